# baseline (speedup 1.0000x reference)
import jax
import jax.numpy as jnp
from jax import lax
from jax.experimental import pallas as pl
from jax.experimental.pallas import tpu as pltpu
from jax.experimental.pallas import tpu_sc as plsc

D = 64
B = 1024
L = 200
LANES = 16

NC = 2
NS = 16
NW = NC * NS
ROUNDS = -(-L // NW)
QT = 256
NQ = B // QT
PLANE = D * B
TV = 1000000

NCHUNK = ROUNDS * NQ
NPIPE = 30
PITCH = 129


def _body(idsT_hbm, table_hbm, pos_hbm, out_hbm,
          idx_all, pos_all, rows0, rows1, rows2, stage0, stage1, stage2,
          sem_i, sem_g, sem_o):
    wid = lax.axis_index("s") * NC + lax.axis_index("c")

    ls = [jnp.where(wid + NW * j < L, wid + NW * j, wid) for j in range(ROUNDS)]

    pre = []
    for j in range(ROUNDS):
        pre.append(pltpu.async_copy(
            idsT_hbm.at[pl.ds(ls[j] * B, B)], idx_all.at[j], sem_i))
        pre.append(pltpu.async_copy(
            pos_hbm.at[pl.ds(ls[j] * D, D)], pos_all.at[pl.ds(j * D, D)], sem_i))
    for h in pre:
        h.wait()

    rows = [rows0, rows1, rows2]
    stages = [stage0, stage1, stage2]

    iot = lax.iota(jnp.int32, LANES)
    div = iot % 8
    dbv = [iot // 8 + 2 * c for c in range(4)]

    def fire_gathers(k, b):
        j = jnp.minimum(k // NQ, ROUNDS - 1)
        q = k % NQ
        buf = rows[b]
        for c in range(QT // 128):
            idx_sl = idx_all.at[j, pl.ds(q * QT + c * 128, 128)]
            pltpu.async_copy(
                table_hbm.at[idx_sl], buf.at[pl.ds(c * 128, 128)], sem_g)

    def drain_gathers(b):
        for c in range(QT // 128):
            pltpu.make_async_copy(
                table_hbm.at[pl.ds(0, 128)],
                rows[b].at[pl.ds(c * 128, 128)], sem_g).wait()

    def compute(k, b):
        j = jnp.minimum(k // NQ, ROUNDS - 1)
        q = k % NQ
        buf = rows[b]
        stg = stages[b]
        pv = [pos_all[pl.ds(j * D + LANES * c, LANES)] for c in range(4)]

        def tbody(t, carry):
            bbv = jnp.full((LANES,), t // 128, jnp.int32)
            biv = jnp.full((LANES,), t % 128, jnp.int32)
            for c in range(4):
                val = buf[t, pl.ds(LANES * c, LANES)]
                plsc.store_scatter(stg, [bbv, dbv[c], div, biv], val + pv[c])
            return carry

        lax.fori_loop(0, QT, tbody, 0)

    def fire_out(k, b):
        j = jnp.minimum(k // NQ, ROUNDS - 1)
        q = k % NQ
        l = jnp.where(wid + NW * j < L, wid + NW * j, wid)
        stg = stages[b]
        for bb in range(2):
            for db in range(8):
                pltpu.async_copy(
                    stg.at[bb, db, :, pl.ds(0, 128)],
                    out_hbm.at[l, db, 2 * q + bb], sem_o)

    def drain_out(b):
        for bb in range(2):
            for db in range(8):
                pltpu.make_async_copy(
                    stages[b].at[bb, db, :, pl.ds(0, 128)],
                    out_hbm.at[0, db, bb], sem_o).wait()

    fire_gathers(jnp.int32(0), 0)
    fire_gathers(jnp.int32(1), 1)
    fire_gathers(jnp.int32(2), 2)

    for b in range(3):
        k = jnp.int32(b)
        drain_gathers(b)
        compute(k, b)
        fire_out(k, b)
        fire_gathers(k + 3, b)

    def step(kk, carry):
        for b in range(3):
            k = 3 * kk + b
            drain_out(b)
            drain_gathers(b)
            compute(k, b)
            fire_out(k, b)
            fire_gathers(k + 3, b)
        return carry

    lax.fori_loop(1, NPIPE // 3, step, 0)

    for b in range(3):
        drain_gathers(b)
        drain_out(b)


def kernel(ids, embed_tokens, pos_embed):
    idsT_flat = ids.T.reshape(B * L).astype(jnp.int32)
    pos_flat = pos_embed.reshape(-1)
    mesh = plsc.VectorSubcoreMesh(core_axis_name="c", subcore_axis_name="s")
    k = pl.kernel(
        _body,
        out_type=jax.ShapeDtypeStruct((L, 8, 8, 8, 128), jnp.float32),
        mesh=mesh,
        scratch_types=[
            pltpu.VMEM((ROUNDS, B), jnp.int32),
            pltpu.VMEM((ROUNDS * D,), jnp.float32),
            pltpu.VMEM((QT, D), jnp.float32),
            pltpu.VMEM((QT, D), jnp.float32),
            pltpu.VMEM((QT, D), jnp.float32),
            pltpu.VMEM((2, 8, 8, PITCH), jnp.float32),
            pltpu.VMEM((2, 8, 8, PITCH), jnp.float32),
            pltpu.VMEM((2, 8, 8, PITCH), jnp.float32),
            pltpu.SemaphoreType.DMA,
            pltpu.SemaphoreType.DMA,
            pltpu.SemaphoreType.DMA,
        ],
        compiler_params=pltpu.CompilerParams(
            use_tc_tiling_on_sc=False, needs_layout_passes=False),
    )
    out = k(idsT_flat, embed_tokens, pos_flat)
    return jnp.transpose(out, (2, 4, 0, 1, 3)).reshape(B, L, D)


# speedup vs baseline: 1.5226x; 1.5226x over previous
import jax
import jax.numpy as jnp
from jax import lax
from jax.experimental import pallas as pl
from jax.experimental.pallas import tpu as pltpu
from jax.experimental.pallas import tpu_sc as plsc
from jax.experimental import layout as jlayout

D = 64
B = 1024
L = 200
LANES = 16

NC = 2
NS = 16
NW = NC * NS
ROUNDS = -(-L // NW)
QT = 256
NQ = B // QT
PLANE = D * B
TV = 1000000

NCHUNK = ROUNDS * NQ
PITCH = 129


def _body(idsT_hbm, table_hbm, pos_hbm, out_hbm,
          idx_all, pos_all, rows0, rows1, stage0, stage1,
          sem_i, sem_g, sem_o):
    wid = lax.axis_index("s") * NC + lax.axis_index("c")

    ls = [jnp.where(wid + NW * j < L, wid + NW * j, wid) for j in range(ROUNDS)]

    pre = []
    for j in range(ROUNDS):
        pre.append(pltpu.async_copy(
            idsT_hbm.at[pl.ds(ls[j] * B, B)], idx_all.at[j], sem_i))
        pre.append(pltpu.async_copy(
            pos_hbm.at[pl.ds(ls[j] * D, D)], pos_all.at[pl.ds(j * D, D)], sem_i))
    for h in pre:
        h.wait()

    rows = [rows0, rows1]
    stages = [stage0, stage1]

    iot = lax.iota(jnp.int32, LANES)
    div = iot % 8
    dbv = [iot // 8 + 2 * c for c in range(4)]

    def fire_gathers(k, b):
        j = jnp.minimum(k // NQ, ROUNDS - 1)
        q = k % NQ
        buf = rows[b]
        for c in range(QT // 128):
            idx_sl = idx_all.at[j, pl.ds(q * QT + c * 128, 128)]
            pltpu.async_copy(
                table_hbm.at[idx_sl], buf.at[pl.ds(c * 128, 128)], sem_g)

    def drain_gathers(b):
        for c in range(QT // 128):
            pltpu.make_async_copy(
                table_hbm.at[pl.ds(0, 128)],
                rows[b].at[pl.ds(c * 128, 128)], sem_g).wait()

    def compute(k, b):
        j = k // NQ
        q = k % NQ
        buf = rows[b]
        stg = stages[b]
        pv = [pos_all[pl.ds(j * D + LANES * c, LANES)] for c in range(4)]

        def tbody(t, carry):
            bbv = jnp.full((LANES,), t // 128, jnp.int32)
            biv = jnp.full((LANES,), t % 128, jnp.int32)
            for c in range(4):
                val = buf[t, pl.ds(LANES * c, LANES)]
                plsc.store_scatter(stg, [bbv, dbv[c], div, biv], val + pv[c])
            return carry

        lax.fori_loop(0, QT, tbody, 0)

    def fire_out(k, b):
        j = k // NQ
        q = k % NQ
        l = jnp.where(wid + NW * j < L, wid + NW * j, wid)
        stg = stages[b]
        for bb in range(2):
            for db in range(8):
                pltpu.async_copy(
                    stg.at[bb, db, :, pl.ds(0, 128)],
                    out_hbm.at[l, db, 2 * q + bb], sem_o)

    def drain_out(b):
        for bb in range(2):
            for db in range(8):
                pltpu.make_async_copy(
                    stages[b].at[bb, db, :, pl.ds(0, 128)],
                    out_hbm.at[0, db, bb], sem_o).wait()

    fire_gathers(jnp.int32(0), 0)
    fire_gathers(jnp.int32(1), 1)

    for b in range(2):
        k = jnp.int32(b)
        drain_gathers(b)
        compute(k, b)
        fire_out(k, b)
        fire_gathers(k + 2, b)

    def step(kk, carry):
        for b in range(2):
            k = 2 * kk + b
            drain_out(b)
            drain_gathers(b)
            compute(k, b)
            fire_out(k, b)
            fire_gathers(k + 2, b)
        return carry

    lax.fori_loop(1, NCHUNK // 2, step, 0)

    for b in range(2):
        drain_gathers(b)
        drain_out(b)


def kernel(ids, embed_tokens, pos_embed):
    idsT_flat = ids.T.reshape(B * L).astype(jnp.int32)
    table_lin = jlayout.with_layout_constraint(
        embed_tokens,
        jlayout.Layout(major_to_minor=(0, 1), tiling=((8,),)))
    pos_flat = pos_embed.reshape(-1)
    mesh = plsc.VectorSubcoreMesh(core_axis_name="c", subcore_axis_name="s")
    k = pl.kernel(
        _body,
        out_type=jax.ShapeDtypeStruct((L, 8, 8, 8, 128), jnp.float32),
        mesh=mesh,
        scratch_types=[
            pltpu.VMEM((ROUNDS, B), jnp.int32),
            pltpu.VMEM((ROUNDS * D,), jnp.float32),
            pltpu.VMEM((QT, D), jnp.float32),
            pltpu.VMEM((QT, D), jnp.float32),
            pltpu.VMEM((2, 8, 8, PITCH), jnp.float32),
            pltpu.VMEM((2, 8, 8, PITCH), jnp.float32),
            pltpu.SemaphoreType.DMA,
            pltpu.SemaphoreType.DMA,
            pltpu.SemaphoreType.DMA,
        ],
        compiler_params=pltpu.CompilerParams(
            use_tc_tiling_on_sc=False, needs_layout_passes=False),
    )
    out = k(idsT_flat, table_lin, pos_flat)
    return jnp.transpose(out, (2, 4, 0, 1, 3)).reshape(B, L, D)
